# baseline (speedup 1.0000x reference)
import functools

import jax
import jax.numpy as jnp
from jax import lax
from jax.experimental import pallas as pl
from jax.experimental.pallas import tpu as pltpu
from jax.experimental.pallas import tpu_sc as plsc

B, T, D = 1, 2048, 768
DFF = 4 * D
N_SHARED = 2
N_PRIVATE = 8
TOP_K = 2

BLK = 128
PADROWS = T * TOP_K + N_PRIVATE * BLK
MAXB = PADROWS // BLK
BF = 1536

NC, NS = 2, 16
NW = NC * NS
C_CHUNK = 32


def _router_kernel(x_ref, wr_ref, idx_ref, w_ref):
    logits = jnp.dot(x_ref[...], wr_ref[...],
                     preferred_element_type=jnp.float32)
    n = logits.shape[-1]
    iota = lax.broadcasted_iota(jnp.int32, logits.shape, 1)
    m1 = jnp.max(logits, axis=-1, keepdims=True)
    a1 = jnp.min(jnp.where(logits == m1, iota, n), axis=-1, keepdims=True)
    masked = jnp.where(iota == a1, -jnp.inf, logits)
    m2 = jnp.max(masked, axis=-1, keepdims=True)
    a2 = jnp.min(jnp.where(masked == m2, iota, n), axis=-1, keepdims=True)
    s1 = 1.0 / (1.0 + jnp.exp(m2 - m1))
    s2 = 1.0 - s1
    idx_ref[...] = jnp.concatenate([a1, a2], axis=1)
    w_ref[...] = jnp.concatenate([s1, s2], axis=1)


def _router(x2, Wr, interpret=False):
    return pl.pallas_call(
        _router_kernel,
        out_shape=(jax.ShapeDtypeStruct((T, TOP_K), jnp.int32),
                   jax.ShapeDtypeStruct((T, TOP_K), jnp.float32)),
        interpret=interpret,
    )(x2, Wr)


def _shared_kernel(x_ref, ws1_ref, ws2_ref, out_ref):
    e = pl.program_id(0)
    dblk = pl.program_id(1)

    @pl.when((e == 0) & (dblk == 0))
    def _():
        out_ref[...] = jnp.zeros_like(out_ref)

    w1 = ws1_ref[0].astype(jnp.bfloat16)
    w2 = ws2_ref[0].astype(jnp.bfloat16)
    CH = T // 4
    for q in range(4):
        sl = pl.ds(q * CH, CH)
        h = jax.nn.gelu(
            jnp.dot(x_ref[sl, :], w1,
                    preferred_element_type=jnp.float32).astype(jnp.bfloat16))
        contrib = jnp.dot(h, w2, preferred_element_type=jnp.float32)
        out_ref[sl, :] += contrib * (1.0 / N_SHARED)


def _shared(x2, Ws1, Ws2, interpret=False):
    return pl.pallas_call(
        _shared_kernel,
        grid=(N_SHARED, DFF // BF),
        in_specs=[
            pl.BlockSpec((T, D), lambda e, d: (0, 0)),
            pl.BlockSpec((1, D, BF), lambda e, d: (e, 0, d)),
            pl.BlockSpec((1, BF, D), lambda e, d: (e, d, 0)),
        ],
        out_specs=pl.BlockSpec((T, D), lambda e, d: (0, 0)),
        out_shape=jax.ShapeDtypeStruct((T, D), jnp.float32),
        interpret=interpret,
    )(x2, Ws1, Ws2)


def _gemm_kernel(meta_ref, tok_ref, x_ref, w1_ref, w2_ref, wsort_ref, out_ref):
    b = pl.program_id(0)
    nact = meta_ref[MAXB]

    @pl.when(b < nact)
    def _():
        tok = tok_ref[0, 0].astype(jnp.int32)
        wall = wsort_ref[0, 0]
        HB = BLK // 2
        xs_list = []
        for q in range(2):
            iota_t = lax.broadcasted_iota(jnp.int32, (HB, T), 1)
            oh = jnp.where(tok[q * HB:(q + 1) * HB][:, None] == iota_t,
                           1.0, 0.0).astype(jnp.bfloat16)
            xs_list.append(
                jnp.dot(oh, x_ref[...],
                        preferred_element_type=jnp.float32).astype(jnp.bfloat16))
        w1 = w1_ref[0].astype(jnp.bfloat16)
        w2 = w2_ref[0].astype(jnp.bfloat16)
        for q in range(2):
            sl = pl.ds(q * HB, HB)
            h = jax.nn.gelu(
                jnp.dot(xs_list[q], w1,
                        preferred_element_type=jnp.float32).astype(jnp.bfloat16))
            o = jnp.dot(h, w2, preferred_element_type=jnp.float32)
            out_ref[sl, :] = o * wall[q * HB:(q + 1) * HB][:, None]


def _grouped_gemm(meta, tok3, x16, Wp1, Wp2, wsort3, interpret=False):
    grid_spec = pltpu.PrefetchScalarGridSpec(
        num_scalar_prefetch=1,
        grid=(MAXB,),
        in_specs=[
            pl.BlockSpec((1, 1, BLK), lambda b, m: (b, 0, 0)),
            pl.BlockSpec((T, D), lambda b, m: (0, 0)),
            pl.BlockSpec((1, D, DFF), lambda b, m: (m[b], 0, 0)),
            pl.BlockSpec((1, DFF, D), lambda b, m: (m[b], 0, 0)),
            pl.BlockSpec((1, 1, BLK), lambda b, m: (b, 0, 0)),
        ],
        out_specs=pl.BlockSpec((BLK, D), lambda b, m: (b, 0)),
    )
    return pl.pallas_call(
        _gemm_kernel,
        grid_spec=grid_spec,
        out_shape=jax.ShapeDtypeStruct((PADROWS, D), jnp.float32),
        interpret=interpret,
    )(meta, tok3, x16, Wp1, Wp2, wsort3)


def _sc_combine_body(shared_hbm, rows_hbm, p0_hbm, p1_hbm, out_hbm,
                     i0_v, i1_v, bs_v, ba_v, bb_v, sem):
    wid = lax.axis_index("s") * NC + lax.axis_index("c")
    base = wid * (T // NW)

    def chunk(ci, carry):
        off = base + ci * C_CHUNK
        pltpu.sync_copy(p0_hbm.at[pl.ds(off, C_CHUNK)], i0_v)
        pltpu.sync_copy(p1_hbm.at[pl.ds(off, C_CHUNK)], i1_v)
        pltpu.sync_copy(shared_hbm.at[pl.ds(off, C_CHUNK)], bs_v)
        pltpu.async_copy(rows_hbm.at[i0_v], ba_v, sem).wait()
        pltpu.async_copy(rows_hbm.at[i1_v], bb_v, sem).wait()

        def row(i, c2):
            for j in range(D // 16):
                sl = pl.ds(j * 16, 16)
                bs_v[i, sl] = bs_v[i, sl] + ba_v[i, sl] + bb_v[i, sl]
            return c2

        lax.fori_loop(0, C_CHUNK, row, 0)
        pltpu.sync_copy(bs_v, out_hbm.at[pl.ds(off, C_CHUNK)])
        return carry

    lax.fori_loop(0, T // NW // C_CHUNK, chunk, 0)


@functools.cache
def _sc_combine():
    return pl.kernel(
        _sc_combine_body,
        out_type=jax.ShapeDtypeStruct((T, D), jnp.float32),
        mesh=plsc.VectorSubcoreMesh(core_axis_name="c", subcore_axis_name="s"),
        scratch_types=[
            pltpu.VMEM((C_CHUNK,), jnp.int32),
            pltpu.VMEM((C_CHUNK,), jnp.int32),
            pltpu.VMEM((C_CHUNK, D), jnp.float32),
            pltpu.VMEM((C_CHUNK, D), jnp.float32),
            pltpu.VMEM((C_CHUNK, D), jnp.float32),
            pltpu.SemaphoreType.DMA,
        ],
    )


def _dispatch_metadata(topk_idx, topk_w):
    e_flat = topk_idx.reshape(T * TOP_K)
    oh = (e_flat[:, None] == jnp.arange(N_PRIVATE)[None, :]).astype(jnp.int32)
    inc = jnp.cumsum(oh, axis=0)
    rank = jnp.sum(inc * oh, axis=1) - 1
    counts = inc[-1]
    padded = ((counts + BLK - 1) // BLK) * BLK
    cum = jnp.cumsum(padded)
    offs = cum - padded
    pos = offs[e_flat] + rank
    nact = cum[-1] // BLK

    tokf = (jnp.arange(T * TOP_K, dtype=jnp.int32) // TOP_K).astype(jnp.float32)
    vals = jnp.stack([tokf, topk_w.reshape(T * TOP_K)], axis=0)
    sorted_pair = jnp.zeros((2, PADROWS), jnp.float32).at[:, pos].set(vals)
    tok_sorted = sorted_pair[0]
    w_sorted = sorted_pair[1]

    bstart = jnp.arange(MAXB, dtype=jnp.int32) * BLK
    bexp = jnp.sum((bstart[:, None] >= cum[None, :]).astype(jnp.int32), axis=1)
    bexp = jnp.minimum(bexp, bexp[jnp.maximum(nact - 1, 0)])
    meta = jnp.concatenate([bexp, nact[None].astype(jnp.int32)])
    return pos, tok_sorted, w_sorted, meta


@jax.jit
def kernel(x, Ws1, Ws2, Wp1, Wp2, Wr):
    x2 = x.reshape(T, D)
    x16 = x2.astype(jnp.bfloat16)
    topk_idx, topk_w = _router(x2, Wr)
    pos, tok_sorted, w_sorted, meta = _dispatch_metadata(topk_idx, topk_w)

    shared_out = _shared(x16, Ws1, Ws2)
    rows = _grouped_gemm(meta, tok_sorted.reshape(MAXB, 1, BLK), x16,
                         Wp1, Wp2, w_sorted.reshape(MAXB, 1, BLK))
    pos2 = pos.reshape(T, TOP_K)
    out = _sc_combine()(shared_out, rows, pos2[:, 0], pos2[:, 1])
    return out.reshape(B, T, D)


# speedup vs baseline: 1.3081x; 1.3081x over previous
import functools

import jax
import jax.numpy as jnp
from jax import lax
from jax.experimental import pallas as pl
from jax.experimental.pallas import tpu as pltpu
from jax.experimental.pallas import tpu_sc as plsc

B, T, D = 1, 2048, 768
DFF = 4 * D
N_SHARED = 2
N_PRIVATE = 8
TOP_K = 2

BLK = 256
PADROWS = T * TOP_K + N_PRIVATE * BLK
MAXB = PADROWS // BLK
BF = 3072

NC, NS = 2, 16
NW = NC * NS
C_CHUNK = 32


def _router_kernel(x_ref, wr_ref, idx_ref, w_ref):
    logits = jnp.dot(x_ref[...], wr_ref[...],
                     preferred_element_type=jnp.float32)
    n = logits.shape[-1]
    iota = lax.broadcasted_iota(jnp.int32, logits.shape, 1)
    m1 = jnp.max(logits, axis=-1, keepdims=True)
    a1 = jnp.min(jnp.where(logits == m1, iota, n), axis=-1, keepdims=True)
    masked = jnp.where(iota == a1, -jnp.inf, logits)
    m2 = jnp.max(masked, axis=-1, keepdims=True)
    a2 = jnp.min(jnp.where(masked == m2, iota, n), axis=-1, keepdims=True)
    s1 = 1.0 / (1.0 + jnp.exp(m2 - m1))
    s2 = 1.0 - s1
    idx_ref[...] = jnp.concatenate([a1, a2], axis=1)
    w_ref[...] = jnp.concatenate([s1, s2], axis=1)


def _router(x2, Wr, interpret=False):
    return pl.pallas_call(
        _router_kernel,
        out_shape=(jax.ShapeDtypeStruct((T, TOP_K), jnp.int32),
                   jax.ShapeDtypeStruct((T, TOP_K), jnp.float32)),
        interpret=interpret,
    )(x2, Wr)


def _shared_kernel(x_ref, ws1_ref, ws2_ref, out_ref):
    e = pl.program_id(0)
    dblk = pl.program_id(1)

    @pl.when((e == 0) & (dblk == 0))
    def _():
        out_ref[...] = jnp.zeros_like(out_ref)

    w1 = ws1_ref[0].astype(jnp.bfloat16)
    w2 = ws2_ref[0].astype(jnp.bfloat16)
    CH = T // 4
    for q in range(4):
        sl = pl.ds(q * CH, CH)
        h = jax.nn.gelu(
            jnp.dot(x_ref[sl, :], w1,
                    preferred_element_type=jnp.float32).astype(jnp.bfloat16))
        contrib = jnp.dot(h, w2, preferred_element_type=jnp.float32)
        out_ref[sl, :] += contrib * (1.0 / N_SHARED)


def _shared(x2, Ws1, Ws2, interpret=False):
    return pl.pallas_call(
        _shared_kernel,
        grid=(N_SHARED, DFF // BF),
        in_specs=[
            pl.BlockSpec((T, D), lambda e, d: (0, 0)),
            pl.BlockSpec((1, D, BF), lambda e, d: (e, 0, d)),
            pl.BlockSpec((1, BF, D), lambda e, d: (e, d, 0)),
        ],
        out_specs=pl.BlockSpec((T, D), lambda e, d: (0, 0)),
        out_shape=jax.ShapeDtypeStruct((T, D), jnp.float32),
        interpret=interpret,
    )(x2, Ws1, Ws2)


def _gemm_kernel(meta_ref, tok_ref, x_ref, w1_ref, w2_ref, wsort_ref, out_ref):
    b = pl.program_id(0)
    nact = meta_ref[MAXB]

    @pl.when(b < nact)
    def _():
        tok = tok_ref[0, 0].astype(jnp.int32)
        wall = wsort_ref[0, 0]
        HB = BLK // 2
        xs_list = []
        for q in range(2):
            iota_t = lax.broadcasted_iota(jnp.int32, (HB, T), 1)
            oh = jnp.where(tok[q * HB:(q + 1) * HB][:, None] == iota_t,
                           1.0, 0.0).astype(jnp.bfloat16)
            xs_list.append(
                jnp.dot(oh, x_ref[...],
                        preferred_element_type=jnp.float32).astype(jnp.bfloat16))
        w1 = w1_ref[0].astype(jnp.bfloat16)
        w2 = w2_ref[0].astype(jnp.bfloat16)
        for q in range(2):
            sl = pl.ds(q * HB, HB)
            h = jax.nn.gelu(
                jnp.dot(xs_list[q], w1,
                        preferred_element_type=jnp.float32).astype(jnp.bfloat16))
            o = jnp.dot(h, w2, preferred_element_type=jnp.float32)
            out_ref[sl, :] = o * wall[q * HB:(q + 1) * HB][:, None]


def _grouped_gemm(meta, tok3, x16, Wp1, Wp2, wsort3, interpret=False):
    grid_spec = pltpu.PrefetchScalarGridSpec(
        num_scalar_prefetch=1,
        grid=(MAXB,),
        in_specs=[
            pl.BlockSpec((1, 1, BLK), lambda b, m: (b, 0, 0)),
            pl.BlockSpec((T, D), lambda b, m: (0, 0)),
            pl.BlockSpec((1, D, DFF), lambda b, m: (m[b], 0, 0)),
            pl.BlockSpec((1, DFF, D), lambda b, m: (m[b], 0, 0)),
            pl.BlockSpec((1, 1, BLK), lambda b, m: (b, 0, 0)),
        ],
        out_specs=pl.BlockSpec((BLK, D), lambda b, m: (b, 0)),
    )
    return pl.pallas_call(
        _gemm_kernel,
        grid_spec=grid_spec,
        out_shape=jax.ShapeDtypeStruct((PADROWS, D), jnp.float32),
        interpret=interpret,
    )(meta, tok3, x16, Wp1, Wp2, wsort3)


def _sc_combine_body(shared_hbm, rows_hbm, p0_hbm, p1_hbm, out_hbm,
                     i0_v, i1_v, bs_v, ba_v, bb_v, sem):
    wid = lax.axis_index("s") * NC + lax.axis_index("c")
    base = wid * (T // NW)

    def chunk(ci, carry):
        off = base + ci * C_CHUNK
        pltpu.sync_copy(p0_hbm.at[pl.ds(off, C_CHUNK)], i0_v)
        pltpu.sync_copy(p1_hbm.at[pl.ds(off, C_CHUNK)], i1_v)
        pltpu.sync_copy(shared_hbm.at[pl.ds(off, C_CHUNK)], bs_v)
        pltpu.async_copy(rows_hbm.at[i0_v], ba_v, sem).wait()
        pltpu.async_copy(rows_hbm.at[i1_v], bb_v, sem).wait()

        def row(i, c2):
            for j in range(D // 16):
                sl = pl.ds(j * 16, 16)
                bs_v[i, sl] = bs_v[i, sl] + ba_v[i, sl] + bb_v[i, sl]
            return c2

        lax.fori_loop(0, C_CHUNK, row, 0)
        pltpu.sync_copy(bs_v, out_hbm.at[pl.ds(off, C_CHUNK)])
        return carry

    lax.fori_loop(0, T // NW // C_CHUNK, chunk, 0)


@functools.cache
def _sc_combine():
    return pl.kernel(
        _sc_combine_body,
        out_type=jax.ShapeDtypeStruct((T, D), jnp.float32),
        mesh=plsc.VectorSubcoreMesh(core_axis_name="c", subcore_axis_name="s"),
        scratch_types=[
            pltpu.VMEM((C_CHUNK,), jnp.int32),
            pltpu.VMEM((C_CHUNK,), jnp.int32),
            pltpu.VMEM((C_CHUNK, D), jnp.float32),
            pltpu.VMEM((C_CHUNK, D), jnp.float32),
            pltpu.VMEM((C_CHUNK, D), jnp.float32),
            pltpu.SemaphoreType.DMA,
        ],
    )


def _dispatch_metadata(topk_idx, topk_w):
    e_flat = topk_idx.reshape(T * TOP_K)
    oh = (e_flat[:, None] == jnp.arange(N_PRIVATE)[None, :]).astype(jnp.int32)
    inc = jnp.cumsum(oh, axis=0)
    rank = jnp.sum(inc * oh, axis=1) - 1
    counts = inc[-1]
    padded = ((counts + BLK - 1) // BLK) * BLK
    cum = jnp.cumsum(padded)
    offs = cum - padded
    pos = offs[e_flat] + rank
    nact = cum[-1] // BLK

    tokf = (jnp.arange(T * TOP_K, dtype=jnp.int32) // TOP_K).astype(jnp.float32)
    vals = jnp.stack([tokf, topk_w.reshape(T * TOP_K)], axis=0)
    sorted_pair = jnp.zeros((2, PADROWS), jnp.float32).at[:, pos].set(vals)
    tok_sorted = sorted_pair[0]
    w_sorted = sorted_pair[1]

    bstart = jnp.arange(MAXB, dtype=jnp.int32) * BLK
    bexp = jnp.sum((bstart[:, None] >= cum[None, :]).astype(jnp.int32), axis=1)
    bexp = jnp.minimum(bexp, bexp[jnp.maximum(nact - 1, 0)])
    meta = jnp.concatenate([bexp, nact[None].astype(jnp.int32)])
    return pos, tok_sorted, w_sorted, meta


@jax.jit
def kernel(x, Ws1, Ws2, Wp1, Wp2, Wr):
    x2 = x.reshape(T, D)
    x16 = x2.astype(jnp.bfloat16)
    topk_idx, topk_w = _router(x2, Wr)
    pos, tok_sorted, w_sorted, meta = _dispatch_metadata(topk_idx, topk_w)

    shared_out = _shared(x16, Ws1, Ws2)
    rows = _grouped_gemm(meta, tok_sorted.reshape(MAXB, 1, BLK), x16,
                         Wp1, Wp2, w_sorted.reshape(MAXB, 1, BLK))
    pos2 = pos.reshape(T, TOP_K)
    out = _sc_combine()(shared_out, rows, pos2[:, 0], pos2[:, 1])
    return out.reshape(B, T, D)


# speedup vs baseline: 1.3339x; 1.0198x over previous
import functools

import jax
import jax.numpy as jnp
from jax import lax
from jax.experimental import pallas as pl
from jax.experimental.pallas import tpu as pltpu
from jax.experimental.pallas import tpu_sc as plsc

B, T, D = 1, 2048, 768
DFF = 4 * D
N_SHARED = 2
N_PRIVATE = 8
TOP_K = 2

BLK = 256
PADROWS = T * TOP_K + N_PRIVATE * BLK
MAXB = PADROWS // BLK
BF = 1536

NC, NS = 2, 16
NW = NC * NS
C_CHUNK = 32


def _router_kernel(x_ref, wr_ref, idx_ref, w_ref):
    logits = jnp.dot(x_ref[...], wr_ref[...],
                     preferred_element_type=jnp.float32)
    n = logits.shape[-1]
    iota = lax.broadcasted_iota(jnp.int32, logits.shape, 1)
    m1 = jnp.max(logits, axis=-1, keepdims=True)
    a1 = jnp.min(jnp.where(logits == m1, iota, n), axis=-1, keepdims=True)
    masked = jnp.where(iota == a1, -jnp.inf, logits)
    m2 = jnp.max(masked, axis=-1, keepdims=True)
    a2 = jnp.min(jnp.where(masked == m2, iota, n), axis=-1, keepdims=True)
    s1 = 1.0 / (1.0 + jnp.exp(m2 - m1))
    s2 = 1.0 - s1
    idx_ref[...] = jnp.concatenate([a1, a2], axis=1)
    w_ref[...] = jnp.concatenate([s1, s2], axis=1)


def _router(x2, Wr, interpret=False):
    return pl.pallas_call(
        _router_kernel,
        out_shape=(jax.ShapeDtypeStruct((T, TOP_K), jnp.int32),
                   jax.ShapeDtypeStruct((T, TOP_K), jnp.float32)),
        interpret=interpret,
    )(x2, Wr)


def _shared_kernel(x_ref, ws1_ref, ws2_ref, out_ref):
    e = pl.program_id(0)
    dblk = pl.program_id(1)

    @pl.when((e == 0) & (dblk == 0))
    def _():
        out_ref[...] = jnp.zeros_like(out_ref)

    w1 = ws1_ref[0].astype(jnp.bfloat16)
    w2 = ws2_ref[0].astype(jnp.bfloat16)
    CH = T // 4
    for q in range(4):
        sl = pl.ds(q * CH, CH)
        h = jax.nn.gelu(
            jnp.dot(x_ref[sl, :], w1,
                    preferred_element_type=jnp.float32).astype(jnp.bfloat16))
        contrib = jnp.dot(h, w2, preferred_element_type=jnp.float32)
        out_ref[sl, :] += contrib * (1.0 / N_SHARED)


def _shared(x2, Ws1, Ws2, interpret=False):
    return pl.pallas_call(
        _shared_kernel,
        grid=(N_SHARED, DFF // BF),
        in_specs=[
            pl.BlockSpec((T, D), lambda e, d: (0, 0)),
            pl.BlockSpec((1, D, BF), lambda e, d: (e, 0, d)),
            pl.BlockSpec((1, BF, D), lambda e, d: (e, d, 0)),
        ],
        out_specs=pl.BlockSpec((T, D), lambda e, d: (0, 0)),
        out_shape=jax.ShapeDtypeStruct((T, D), jnp.float32),
        interpret=interpret,
    )(x2, Ws1, Ws2)


def _gemm_kernel(meta_ref, tok_ref, x_ref, w1_ref, w2_ref, wsort_ref, out_ref):
    b = pl.program_id(0)
    nact = meta_ref[MAXB]

    @pl.when(b < nact)
    def _():
        tok = tok_ref[0, 0].astype(jnp.int32)
        wall = wsort_ref[0, 0]
        HB = BLK // 2
        xs_list = []
        for q in range(2):
            iota_t = lax.broadcasted_iota(jnp.int32, (HB, T), 1)
            oh = jnp.where(tok[q * HB:(q + 1) * HB][:, None] == iota_t,
                           1.0, 0.0).astype(jnp.bfloat16)
            xs_list.append(
                jnp.dot(oh, x_ref[...],
                        preferred_element_type=jnp.float32).astype(jnp.bfloat16))
        w1 = w1_ref[0].astype(jnp.bfloat16)
        w2 = w2_ref[0].astype(jnp.bfloat16)
        for q in range(2):
            sl = pl.ds(q * HB, HB)
            h = jax.nn.gelu(
                jnp.dot(xs_list[q], w1,
                        preferred_element_type=jnp.float32).astype(jnp.bfloat16))
            o = jnp.dot(h, w2, preferred_element_type=jnp.float32)
            out_ref[sl, :] = o * wall[q * HB:(q + 1) * HB][:, None]


def _grouped_gemm(meta, tok3, x16, Wp1, Wp2, wsort3, interpret=False):
    grid_spec = pltpu.PrefetchScalarGridSpec(
        num_scalar_prefetch=1,
        grid=(MAXB,),
        in_specs=[
            pl.BlockSpec((1, 1, BLK), lambda b, m: (b, 0, 0)),
            pl.BlockSpec((T, D), lambda b, m: (0, 0)),
            pl.BlockSpec((1, D, DFF), lambda b, m: (m[b], 0, 0)),
            pl.BlockSpec((1, DFF, D), lambda b, m: (m[b], 0, 0)),
            pl.BlockSpec((1, 1, BLK), lambda b, m: (b, 0, 0)),
        ],
        out_specs=pl.BlockSpec((BLK, D), lambda b, m: (b, 0)),
    )
    return pl.pallas_call(
        _gemm_kernel,
        grid_spec=grid_spec,
        out_shape=jax.ShapeDtypeStruct((PADROWS, D), jnp.float32),
        interpret=interpret,
    )(meta, tok3, x16, Wp1, Wp2, wsort3)


def _sc_combine_body(shared_hbm, rows_hbm, p0_hbm, p1_hbm, out_hbm,
                     i0_v, i1_v, bs_v, ba_v, bb_v, sem, sem2, sem3):
    wid = lax.axis_index("s") * NC + lax.axis_index("c")
    base = wid * (T // NW)

    def chunk(ci, carry):
        off = base + ci * C_CHUNK
        pltpu.sync_copy(p0_hbm.at[pl.ds(off, C_CHUNK)], i0_v)
        pltpu.sync_copy(p1_hbm.at[pl.ds(off, C_CHUNK)], i1_v)
        cps = pltpu.async_copy(shared_hbm.at[pl.ds(off, C_CHUNK)], bs_v, sem3)
        cpa = pltpu.async_copy(rows_hbm.at[i0_v], ba_v, sem)
        cpb = pltpu.async_copy(rows_hbm.at[i1_v], bb_v, sem2)
        cps.wait()
        cpa.wait()
        cpb.wait()

        def row(i, c2):
            for j in range(D // 16):
                sl = pl.ds(j * 16, 16)
                bs_v[i, sl] = bs_v[i, sl] + ba_v[i, sl] + bb_v[i, sl]
            return c2

        lax.fori_loop(0, C_CHUNK, row, 0)
        pltpu.sync_copy(bs_v, out_hbm.at[pl.ds(off, C_CHUNK)])
        return carry

    lax.fori_loop(0, T // NW // C_CHUNK, chunk, 0)


@functools.cache
def _sc_combine():
    return pl.kernel(
        _sc_combine_body,
        out_type=jax.ShapeDtypeStruct((T, D), jnp.float32),
        mesh=plsc.VectorSubcoreMesh(core_axis_name="c", subcore_axis_name="s"),
        scratch_types=[
            pltpu.VMEM((C_CHUNK,), jnp.int32),
            pltpu.VMEM((C_CHUNK,), jnp.int32),
            pltpu.VMEM((C_CHUNK, D), jnp.float32),
            pltpu.VMEM((C_CHUNK, D), jnp.float32),
            pltpu.VMEM((C_CHUNK, D), jnp.float32),
            pltpu.SemaphoreType.DMA,
            pltpu.SemaphoreType.DMA,
            pltpu.SemaphoreType.DMA,
        ],
    )


def _dispatch_metadata(topk_idx, topk_w):
    e_flat = topk_idx.reshape(T * TOP_K)
    oh = (e_flat[:, None] == jnp.arange(N_PRIVATE)[None, :]).astype(jnp.int32)
    inc = jnp.cumsum(oh, axis=0)
    rank = jnp.sum(inc * oh, axis=1) - 1
    counts = inc[-1]
    padded = ((counts + BLK - 1) // BLK) * BLK
    cum = jnp.cumsum(padded)
    offs = cum - padded
    pos = offs[e_flat] + rank
    nact = cum[-1] // BLK

    tokf = (jnp.arange(T * TOP_K, dtype=jnp.int32) // TOP_K).astype(jnp.float32)
    vals = jnp.stack([tokf, topk_w.reshape(T * TOP_K)], axis=0)
    sorted_pair = jnp.zeros((2, PADROWS), jnp.float32).at[:, pos].set(vals)
    tok_sorted = sorted_pair[0]
    w_sorted = sorted_pair[1]

    bstart = jnp.arange(MAXB, dtype=jnp.int32) * BLK
    bexp = jnp.sum((bstart[:, None] >= cum[None, :]).astype(jnp.int32), axis=1)
    bexp = jnp.minimum(bexp, bexp[jnp.maximum(nact - 1, 0)])
    meta = jnp.concatenate([bexp, nact[None].astype(jnp.int32)])
    return pos, tok_sorted, w_sorted, meta


@jax.jit
def kernel(x, Ws1, Ws2, Wp1, Wp2, Wr):
    x2 = x.reshape(T, D)
    x16 = x2.astype(jnp.bfloat16)
    topk_idx, topk_w = _router(x2, Wr)
    pos, tok_sorted, w_sorted, meta = _dispatch_metadata(topk_idx, topk_w)

    shared_out = _shared(x16, Ws1, Ws2)
    rows = _grouped_gemm(meta, tok_sorted.reshape(MAXB, 1, BLK), x16,
                         Wp1, Wp2, w_sorted.reshape(MAXB, 1, BLK))
    pos2 = pos.reshape(T, TOP_K)
    out = _sc_combine()(shared_out, rows, pos2[:, 0], pos2[:, 1])
    return out.reshape(B, T, D)
